# baseline (speedup 1.0000x reference)
import jax
import jax.numpy as jnp
from jax.experimental import pallas as pl
from jax.experimental.pallas import tpu as pltpu

B, N = 16, 4096
X_DIM, Y_DIM, H_DIM, R_DIM = 16, 16, 64, 64
K = 8
CHUNK = N // K


def _body(x_ref, y_ref, m_ref, w1_ref, b1_ref, w2_ref, b2_ref, w3_ref, b3_ref,
          out_ref, acc_ref, cnt_ref):
    k = pl.program_id(1)
    xb = x_ref[0]
    yb = y_ref[0]
    m = m_ref[0]
    nn_in = jnp.concatenate([xb, yb], axis=1)
    h = jnp.dot(nn_in, w1_ref[...], preferred_element_type=jnp.float32)
    h = jnp.maximum(h + b1_ref[...], 0.0)
    h2 = jnp.dot(h, w2_ref[...], preferred_element_type=jnp.float32)
    h2 = jnp.maximum(h2 + b2_ref[...], 0.0)
    s = jnp.sum(h2 * m, axis=0, keepdims=True)
    cnt = jnp.sum(m, axis=0, keepdims=True)

    @pl.when(k == 0)
    def _init():
        acc_ref[...] = s
        cnt_ref[...] = cnt

    @pl.when(k > 0)
    def _accum():
        acc_ref[...] += s
        cnt_ref[...] += cnt

    @pl.when(k == K - 1)
    def _finish():
        c = cnt_ref[0, 0]
        r = jnp.dot(acc_ref[...], w3_ref[...],
                    preferred_element_type=jnp.float32)
        r = r + c * b3_ref[...]
        out_ref[0] = r / jnp.maximum(c, 1.0)


def kernel(x, y, mask, W1, b1, W2, b2, W3, b3):
    mf = mask.astype(jnp.float32).reshape(B, N, 1)
    b1r = b1.reshape(1, H_DIM)
    b2r = b2.reshape(1, H_DIM)
    b3r = b3.reshape(1, R_DIM)

    out = pl.pallas_call(
        _body,
        grid=(B, K),
        in_specs=[
            pl.BlockSpec((1, CHUNK, X_DIM), lambda b, k: (b, k, 0)),
            pl.BlockSpec((1, CHUNK, Y_DIM), lambda b, k: (b, k, 0)),
            pl.BlockSpec((1, CHUNK, 1), lambda b, k: (b, k, 0)),
            pl.BlockSpec((X_DIM + Y_DIM, H_DIM), lambda b, k: (0, 0)),
            pl.BlockSpec((1, H_DIM), lambda b, k: (0, 0)),
            pl.BlockSpec((H_DIM, H_DIM), lambda b, k: (0, 0)),
            pl.BlockSpec((1, H_DIM), lambda b, k: (0, 0)),
            pl.BlockSpec((H_DIM, R_DIM), lambda b, k: (0, 0)),
            pl.BlockSpec((1, R_DIM), lambda b, k: (0, 0)),
        ],
        out_specs=pl.BlockSpec((1, 1, R_DIM), lambda b, k: (b, 0, 0)),
        out_shape=jax.ShapeDtypeStruct((B, 1, R_DIM), jnp.float32),
        scratch_shapes=[
            pltpu.VMEM((1, H_DIM), jnp.float32),
            pltpu.VMEM((1, 1), jnp.float32),
        ],
        compiler_params=pltpu.CompilerParams(
            dimension_semantics=("arbitrary", "arbitrary"),
        ),
    )(x, y, mf, W1, b1r, W2, b2r, W3, b3r)
    return out.reshape(B, R_DIM)


# speedup vs baseline: 1.4720x; 1.4720x over previous
import jax
import jax.numpy as jnp
from jax.experimental import pallas as pl
from jax.experimental.pallas import tpu as pltpu

B, N = 16, 4096
X_DIM, Y_DIM, H_DIM, R_DIM = 16, 16, 64, 64
PACK = 128 // X_DIM
PROWS = N // PACK
NPAIR = PACK // 2


def _body(x_ref, y_ref, m_ref, w1x_ref, w1y_ref, b1_ref, w2_ref, b2_ref,
          w3_ref, b3_ref, e_ref, out_ref):
    px = x_ref[0]
    py = y_ref[0]
    mp = m_ref[0]
    h = jnp.dot(px, w1x_ref[...], preferred_element_type=jnp.float32)
    h = h + jnp.dot(py, w1y_ref[...], preferred_element_type=jnp.float32)
    h = jnp.maximum(h + b1_ref[...], 0.0)
    mexp = jnp.dot(mp, e_ref[...], preferred_element_type=jnp.float32)
    acc = jnp.zeros((1, 2 * H_DIM), dtype=jnp.float32)
    for p in range(NPAIR):
        g = h[:, 2 * H_DIM * p:2 * H_DIM * (p + 1)]
        h2 = jnp.dot(g, w2_ref[...], preferred_element_type=jnp.float32)
        h2 = jnp.maximum(h2 + b2_ref[...], 0.0)
        mm = mexp[:, 2 * H_DIM * p:2 * H_DIM * (p + 1)]
        acc = acc + jnp.sum(h2 * mm, axis=0, keepdims=True)
    s = acc[:, :H_DIM] + acc[:, H_DIM:]
    cnt = jnp.sum(mp)
    r = jnp.dot(s, w3_ref[...], preferred_element_type=jnp.float32)
    r = r + cnt * b3_ref[...]
    out_ref[0] = r / jnp.maximum(cnt, 1.0)


def kernel(x, y, mask, W1, b1, W2, b2, W3, b3):
    xd = x.reshape(B, PROWS, 128)
    yd = y.reshape(B, PROWS, 128)
    mp = mask.astype(jnp.float32).reshape(B, PROWS, PACK)
    eye8 = jnp.eye(PACK, dtype=jnp.float32)
    w1x_bd = jnp.kron(eye8, W1[:X_DIM])
    w1y_bd = jnp.kron(eye8, W1[X_DIM:])
    w2_bd = jnp.kron(jnp.eye(2, dtype=jnp.float32), W2)
    e_mat = jnp.kron(eye8, jnp.ones((1, H_DIM), jnp.float32))
    b1t = jnp.tile(b1, PACK).reshape(1, PACK * H_DIM)
    b2t = jnp.tile(b2, 2).reshape(1, 2 * H_DIM)
    b3r = b3.reshape(1, R_DIM)

    out = pl.pallas_call(
        _body,
        grid=(B,),
        in_specs=[
            pl.BlockSpec((1, PROWS, 128), lambda b: (b, 0, 0)),
            pl.BlockSpec((1, PROWS, 128), lambda b: (b, 0, 0)),
            pl.BlockSpec((1, PROWS, PACK), lambda b: (b, 0, 0)),
            pl.BlockSpec((128, PACK * H_DIM), lambda b: (0, 0)),
            pl.BlockSpec((128, PACK * H_DIM), lambda b: (0, 0)),
            pl.BlockSpec((1, PACK * H_DIM), lambda b: (0, 0)),
            pl.BlockSpec((2 * H_DIM, 2 * H_DIM), lambda b: (0, 0)),
            pl.BlockSpec((1, 2 * H_DIM), lambda b: (0, 0)),
            pl.BlockSpec((H_DIM, R_DIM), lambda b: (0, 0)),
            pl.BlockSpec((1, R_DIM), lambda b: (0, 0)),
            pl.BlockSpec((PACK, PACK * H_DIM), lambda b: (0, 0)),
        ],
        out_specs=pl.BlockSpec((1, 1, R_DIM), lambda b: (b, 0, 0)),
        out_shape=jax.ShapeDtypeStruct((B, 1, R_DIM), jnp.float32),
        compiler_params=pltpu.CompilerParams(
            dimension_semantics=("arbitrary",),
        ),
    )(xd, yd, mp, w1x_bd, w1y_bd, b1t, w2_bd, b2t, W3, b3r, e_mat)
    return out.reshape(B, R_DIM)


# speedup vs baseline: 1.5382x; 1.0450x over previous
import jax
import jax.numpy as jnp
from jax import lax
from jax.experimental import pallas as pl
from jax.experimental.pallas import tpu as pltpu

B, N = 16, 4096
X_DIM, Y_DIM, H_DIM, R_DIM = 16, 16, 64, 64
PACK = 128 // X_DIM
PROWS = N // PACK
NPAIR = PACK // 2
NSPLIT = 4
QROWS = PROWS // NSPLIT


def _bd_mask(rows, cols, rblk, cblk):
    ri = lax.broadcasted_iota(jnp.int32, (rows, cols), 0) // rblk
    ci = lax.broadcasted_iota(jnp.int32, (rows, cols), 1) // cblk
    return (ri == ci).astype(jnp.float32)


def _body(x0, x1, x2, x3, y0, y1, y2, y3, m_ref, w1_ref, b1_ref, w2_ref,
          b2_ref, w3_ref, b3_ref, out_ref):
    w1 = w1_ref[...]
    w1x_bd = jnp.tile(w1[:X_DIM], (PACK, PACK)) * _bd_mask(128, 512, 16, 64)
    w1y_bd = jnp.tile(w1[X_DIM:], (PACK, PACK)) * _bd_mask(128, 512, 16, 64)
    w2_bd = jnp.tile(w2_ref[...], (2, 2)) * _bd_mask(128, 128, 64, 64)
    b1t = jnp.tile(b1_ref[...], (1, PACK))
    b2t = jnp.tile(b2_ref[...], (1, 2))
    e_mat = _bd_mask(PACK, PACK * H_DIM, 1, H_DIM)

    px = jnp.concatenate([x0[0], x1[0], x2[0], x3[0]], axis=0)
    py = jnp.concatenate([y0[0], y1[0], y2[0], y3[0]], axis=0)
    mp = m_ref[0]

    h = jnp.dot(px, w1x_bd, preferred_element_type=jnp.float32)
    h = h + jnp.dot(py, w1y_bd, preferred_element_type=jnp.float32)
    h = jnp.maximum(h + b1t, 0.0)
    mexp = jnp.dot(mp, e_mat, preferred_element_type=jnp.float32)
    acc = jnp.zeros((1, 2 * H_DIM), dtype=jnp.float32)
    for p in range(NPAIR):
        g = h[:, 2 * H_DIM * p:2 * H_DIM * (p + 1)]
        h2 = jnp.dot(g, w2_bd, preferred_element_type=jnp.float32)
        h2 = jnp.maximum(h2 + b2t, 0.0)
        mm = mexp[:, 2 * H_DIM * p:2 * H_DIM * (p + 1)]
        acc = acc + jnp.sum(h2 * mm, axis=0, keepdims=True)
    s = acc[:, :H_DIM] + acc[:, H_DIM:]
    cnt = jnp.sum(mp)
    r = jnp.dot(s, w3_ref[...], preferred_element_type=jnp.float32)
    r = r + cnt * b3_ref[...]
    out_ref[0] = r / jnp.maximum(cnt, 1.0)


def kernel(x, y, mask, W1, b1, W2, b2, W3, b3):
    xd = x.reshape(B, PROWS, 128)
    yd = y.reshape(B, PROWS, 128)
    mp = mask.astype(jnp.float32).reshape(B, PROWS, PACK)
    b1r = b1.reshape(1, H_DIM)
    b2r = b2.reshape(1, H_DIM)
    b3r = b3.reshape(1, R_DIM)

    def qspec(q):
        return pl.BlockSpec((1, QROWS, 128), lambda b, q=q: (b, q, 0))

    out = pl.pallas_call(
        _body,
        grid=(B,),
        in_specs=[
            qspec(0), qspec(1), qspec(2), qspec(3),
            qspec(0), qspec(1), qspec(2), qspec(3),
            pl.BlockSpec((1, PROWS, PACK), lambda b: (b, 0, 0)),
            pl.BlockSpec((X_DIM + Y_DIM, H_DIM), lambda b: (0, 0)),
            pl.BlockSpec((1, H_DIM), lambda b: (0, 0)),
            pl.BlockSpec((H_DIM, H_DIM), lambda b: (0, 0)),
            pl.BlockSpec((1, H_DIM), lambda b: (0, 0)),
            pl.BlockSpec((H_DIM, R_DIM), lambda b: (0, 0)),
            pl.BlockSpec((1, R_DIM), lambda b: (0, 0)),
        ],
        out_specs=pl.BlockSpec((1, 1, R_DIM), lambda b: (b, 0, 0)),
        out_shape=jax.ShapeDtypeStruct((B, 1, R_DIM), jnp.float32),
        compiler_params=pltpu.CompilerParams(
            dimension_semantics=("arbitrary",),
        ),
    )(xd, xd, xd, xd, yd, yd, yd, yd, mp, W1, b1r, W2, b2r, W3, b3r)
    return out.reshape(B, R_DIM)
